# baseline (speedup 1.0000x reference)
import functools

import jax
import jax.numpy as jnp
from jax.experimental import pallas as pl
from jax.experimental.pallas import tpu as pltpu

NUM_EMBEDDINGS = 1024
EMBEDDING_DIM = 64
COMMITMENT_COST = 0.25
N_ROWS = 8192
BATCH_PER_STEP = 1
BLOCK_ROWS = 1024 * BATCH_PER_STEP
N_BLOCKS = N_ROWS // BLOCK_ROWS


def _vq_kernel(x_ref, emb_ref, enc_ref, q_ref, loss_ref, perp_ref,
               loss_acc, hist_acc):
    step = pl.program_id(0)
    x_cf = jnp.concatenate([x_ref[i] for i in range(BATCH_PER_STEP)], axis=1)
    emb = emb_ref[:]

    x2 = jnp.sum(x_cf * x_cf, axis=0, keepdims=True)
    e2 = jnp.sum(emb * emb, axis=1)[:, None]
    m2 = jax.lax.dot_general(emb + emb, x_cf, (((1,), (0,)), ((), ())),
                             preferred_element_type=jnp.float32)
    d = (x2 + e2) - m2

    idx = jnp.argmin(d, axis=0)
    idx_col = idx[:, None]
    iota = jax.lax.broadcasted_iota(jnp.int32, (BLOCK_ROWS, NUM_EMBEDDINGS), 1)
    onehot = (idx_col == iota).astype(jnp.float32)
    enc_ref[:] = onehot

    q = jax.lax.dot_general(emb, onehot, (((0,), (1,)), ((), ())),
                            preferred_element_type=jnp.float32)
    qst = x_cf + (q - x_cf)
    for i in range(BATCH_PER_STEP):
        q_ref[i] = qst[:, i * 1024:(i + 1) * 1024]

    @pl.when(step == 0)
    def _init():
        loss_acc[:] = jnp.zeros_like(loss_acc)
        hist_acc[:] = jnp.zeros_like(hist_acc)

    loss_acc[:] += jnp.sum((q - x_cf) ** 2).reshape(1, 1)
    hist_acc[:] += jax.lax.dot_general(
        jnp.ones((1, BLOCK_ROWS), jnp.float32), onehot,
        (((1,), (0,)), ((), ())), preferred_element_type=jnp.float32)

    @pl.when(step == N_BLOCKS - 1)
    def _fin():
        loss_ref[:] = COMMITMENT_COST * loss_acc[:] / (N_ROWS * EMBEDDING_DIM)
        p = hist_acc[:] / float(N_ROWS)
        perp_ref[:] = jnp.exp(-jnp.sum(p * jnp.log(p + 1e-10))).reshape(1, 1)


@functools.partial(jax.jit, static_argnames=("interpret",))
def kernel(inputs, emb_weight, interpret=False):
    x_cf = inputs.reshape(8, EMBEDDING_DIM, 1024)

    enc, q_st, loss, perp = pl.pallas_call(
        _vq_kernel,
        grid=(N_BLOCKS,),
        in_specs=[
            pl.BlockSpec((BATCH_PER_STEP, EMBEDDING_DIM, 1024),
                         lambda i: (i, 0, 0)),
            pl.BlockSpec((NUM_EMBEDDINGS, EMBEDDING_DIM), lambda i: (0, 0)),
        ],
        out_specs=[
            pl.BlockSpec((BLOCK_ROWS, NUM_EMBEDDINGS), lambda i: (i, 0)),
            pl.BlockSpec((BATCH_PER_STEP, EMBEDDING_DIM, 1024),
                         lambda i: (i, 0, 0)),
            pl.BlockSpec((1, 1), lambda i: (0, 0)),
            pl.BlockSpec((1, 1), lambda i: (0, 0)),
        ],
        out_shape=[
            jax.ShapeDtypeStruct((N_ROWS, NUM_EMBEDDINGS), jnp.float32),
            jax.ShapeDtypeStruct((8, EMBEDDING_DIM, 1024), jnp.float32),
            jax.ShapeDtypeStruct((1, 1), jnp.float32),
            jax.ShapeDtypeStruct((1, 1), jnp.float32),
        ],
        scratch_shapes=[
            pltpu.VMEM((1, 1), jnp.float32),
            pltpu.VMEM((1, NUM_EMBEDDINGS), jnp.float32),
        ],
        interpret=interpret,
    )(x_cf, emb_weight)

    quantized_out = q_st.reshape(8, EMBEDDING_DIM, 32, 32)
    return (loss[0, 0], quantized_out, perp[0, 0], enc)


# speedup vs baseline: 1.0204x; 1.0204x over previous
import functools

import jax
import jax.numpy as jnp
from jax.experimental import pallas as pl
from jax.experimental.pallas import tpu as pltpu

NUM_EMBEDDINGS = 1024
EMBEDDING_DIM = 64
COMMITMENT_COST = 0.25
N_ROWS = 8192
BATCH_PER_STEP = 2
BLOCK_ROWS = 1024 * BATCH_PER_STEP
N_BLOCKS = N_ROWS // BLOCK_ROWS


def _vq_kernel(x_ref, emb_ref, enc_ref, q_ref, loss_ref, perp_ref,
               loss_acc, hist_acc):
    step = pl.program_id(0)
    x_cf = jnp.concatenate([x_ref[i] for i in range(BATCH_PER_STEP)], axis=1)
    emb = emb_ref[:]

    x2 = jnp.sum(x_cf * x_cf, axis=0, keepdims=True)
    e2 = jnp.sum(emb * emb, axis=1)[:, None]
    m2 = jax.lax.dot_general(emb + emb, x_cf, (((1,), (0,)), ((), ())),
                             preferred_element_type=jnp.float32)
    d = (x2 + e2) - m2

    idx = jnp.argmin(d, axis=0)
    idx_col = idx[:, None]
    iota = jax.lax.broadcasted_iota(jnp.int32, (BLOCK_ROWS, NUM_EMBEDDINGS), 1)
    onehot = (idx_col == iota).astype(jnp.float32)
    enc_ref[:] = onehot

    q = jax.lax.dot_general(emb, onehot, (((0,), (1,)), ((), ())),
                            preferred_element_type=jnp.float32)
    qst = x_cf + (q - x_cf)
    for i in range(BATCH_PER_STEP):
        q_ref[i] = qst[:, i * 1024:(i + 1) * 1024]

    @pl.when(step == 0)
    def _init():
        loss_acc[:] = jnp.zeros_like(loss_acc)
        hist_acc[:] = jnp.zeros_like(hist_acc)

    loss_acc[:] += jnp.sum((q - x_cf) ** 2).reshape(1, 1)
    hist_acc[:] += jax.lax.dot_general(
        jnp.ones((1, BLOCK_ROWS), jnp.float32), onehot,
        (((1,), (0,)), ((), ())), preferred_element_type=jnp.float32)

    @pl.when(step == N_BLOCKS - 1)
    def _fin():
        loss_ref[:] = COMMITMENT_COST * loss_acc[:] / (N_ROWS * EMBEDDING_DIM)
        p = hist_acc[:] / float(N_ROWS)
        perp_ref[:] = jnp.exp(-jnp.sum(p * jnp.log(p + 1e-10))).reshape(1, 1)


@functools.partial(jax.jit, static_argnames=("interpret",))
def kernel(inputs, emb_weight, interpret=False):
    x_cf = inputs.reshape(8, EMBEDDING_DIM, 1024)

    enc, q_st, loss, perp = pl.pallas_call(
        _vq_kernel,
        grid=(N_BLOCKS,),
        in_specs=[
            pl.BlockSpec((BATCH_PER_STEP, EMBEDDING_DIM, 1024),
                         lambda i: (i, 0, 0)),
            pl.BlockSpec((NUM_EMBEDDINGS, EMBEDDING_DIM), lambda i: (0, 0)),
        ],
        out_specs=[
            pl.BlockSpec((BLOCK_ROWS, NUM_EMBEDDINGS), lambda i: (i, 0)),
            pl.BlockSpec((BATCH_PER_STEP, EMBEDDING_DIM, 1024),
                         lambda i: (i, 0, 0)),
            pl.BlockSpec((1, 1), lambda i: (0, 0)),
            pl.BlockSpec((1, 1), lambda i: (0, 0)),
        ],
        out_shape=[
            jax.ShapeDtypeStruct((N_ROWS, NUM_EMBEDDINGS), jnp.float32),
            jax.ShapeDtypeStruct((8, EMBEDDING_DIM, 1024), jnp.float32),
            jax.ShapeDtypeStruct((1, 1), jnp.float32),
            jax.ShapeDtypeStruct((1, 1), jnp.float32),
        ],
        scratch_shapes=[
            pltpu.VMEM((1, 1), jnp.float32),
            pltpu.VMEM((1, NUM_EMBEDDINGS), jnp.float32),
        ],
        interpret=interpret,
    )(x_cf, emb_weight)

    quantized_out = q_st.reshape(8, EMBEDDING_DIM, 32, 32)
    return (loss[0, 0], quantized_out, perp[0, 0], enc)


# speedup vs baseline: 1.0225x; 1.0020x over previous
import functools

import jax
import jax.numpy as jnp
from jax.experimental import pallas as pl
from jax.experimental.pallas import tpu as pltpu

NUM_EMBEDDINGS = 1024
EMBEDDING_DIM = 64
COMMITMENT_COST = 0.25
N_ROWS = 8192
BATCH_PER_STEP = 2
BLOCK_ROWS = 1024 * BATCH_PER_STEP
N_BLOCKS = N_ROWS // BLOCK_ROWS


def _vq_kernel(x_ref, emb_ref, enc_ref, q_ref, loss_ref, perp_ref,
               loss_acc, hist_acc):
    step = pl.program_id(0)
    x_cf = jnp.concatenate([x_ref[i] for i in range(BATCH_PER_STEP)], axis=1)
    emb = emb_ref[:]

    x2 = jnp.sum(x_cf * x_cf, axis=0, keepdims=True)
    e2 = jnp.sum(emb * emb, axis=1)[:, None]
    m2 = jax.lax.dot_general(emb + emb, x_cf, (((1,), (0,)), ((), ())),
                             preferred_element_type=jnp.float32)
    d = (x2 + e2) - m2

    idx = jnp.argmin(d, axis=0)
    idx_col = idx[:, None]
    iota = jax.lax.broadcasted_iota(jnp.int32, (BLOCK_ROWS, NUM_EMBEDDINGS), 1)
    onehot = (idx_col == iota).astype(jnp.float32)
    enc_ref[:] = onehot

    q = jax.lax.dot_general(emb, onehot, (((0,), (1,)), ((), ())),
                            preferred_element_type=jnp.float32)
    qst = x_cf + (q - x_cf)
    for i in range(BATCH_PER_STEP):
        q_ref[i] = qst[:, i * 1024:(i + 1) * 1024]

    @pl.when(step == 0)
    def _init():
        loss_acc[:] = jnp.zeros_like(loss_acc)
        hist_acc[:] = jnp.zeros_like(hist_acc)

    loss_acc[:] += jnp.sum((q - x_cf) ** 2).reshape(1, 1)
    hist_acc[:] += jax.lax.dot_general(
        jnp.ones((1, BLOCK_ROWS), jnp.float32), onehot,
        (((1,), (0,)), ((), ())), preferred_element_type=jnp.float32)

    @pl.when(step == N_BLOCKS - 1)
    def _fin():
        loss_ref[:] = COMMITMENT_COST * loss_acc[:] / (N_ROWS * EMBEDDING_DIM)
        p = hist_acc[:] / float(N_ROWS)
        perp_ref[:] = jnp.exp(-jnp.sum(p * jnp.log(p + 1e-10))).reshape(1, 1)


@functools.partial(jax.jit, static_argnames=("interpret",))
def kernel(inputs, emb_weight, interpret=False):
    x_cf = inputs.reshape(8, EMBEDDING_DIM, 1024)

    enc, q_st, loss, perp = pl.pallas_call(
        _vq_kernel,
        grid=(N_BLOCKS,),
        in_specs=[
            pl.BlockSpec((BATCH_PER_STEP, EMBEDDING_DIM, 1024),
                         lambda i: (i, 0, 0)),
            pl.BlockSpec((NUM_EMBEDDINGS, EMBEDDING_DIM), lambda i: (0, 0)),
        ],
        out_specs=[
            pl.BlockSpec((BLOCK_ROWS, NUM_EMBEDDINGS), lambda i: (i, 0)),
            pl.BlockSpec((BATCH_PER_STEP, EMBEDDING_DIM, 1024),
                         lambda i: (i, 0, 0)),
            pl.BlockSpec((1, 1), lambda i: (0, 0)),
            pl.BlockSpec((1, 1), lambda i: (0, 0)),
        ],
        out_shape=[
            jax.ShapeDtypeStruct((N_ROWS, NUM_EMBEDDINGS), jnp.float32),
            jax.ShapeDtypeStruct((8, EMBEDDING_DIM, 1024), jnp.float32),
            jax.ShapeDtypeStruct((1, 1), jnp.float32),
            jax.ShapeDtypeStruct((1, 1), jnp.float32),
        ],
        scratch_shapes=[
            pltpu.VMEM((1, 1), jnp.float32),
            pltpu.VMEM((1, NUM_EMBEDDINGS), jnp.float32),
        ],
        compiler_params=pltpu.CompilerParams(
            vmem_limit_bytes=100 * 1024 * 1024),
        interpret=interpret,
    )(x_cf, emb_weight)

    quantized_out = q_st.reshape(8, EMBEDDING_DIM, 32, 32)
    return (loss[0, 0], quantized_out, perp[0, 0], enc)
